# baseline (speedup 1.0000x reference)
import functools

import jax
import jax.numpy as jnp
from jax import lax
from jax.experimental import pallas as pl
from jax.experimental.pallas import tpu as pltpu
from jax.experimental.pallas import tpu_sc as plsc

DIM_EMB = 32
DIM_HID = 64
DIM_OUT = 2

NUM_SC = 2
NUM_SUBCORES = 16
NW = NUM_SC * NUM_SUBCORES
CHUNK = 128
GROUP = 8
GROUP_ROWS = GROUP * CHUNK


def _make_gather(n_rows: int, d: int):
    assert n_rows % (NW * GROUP_ROWS) == 0
    rows_per_tile = n_rows // NW
    chunks_per_tile = rows_per_tile // CHUNK
    groups_per_tile = chunks_per_tile // GROUP

    mesh = plsc.VectorSubcoreMesh(core_axis_name="c", subcore_axis_name="s")

    @functools.partial(
        pl.kernel,
        mesh=mesh,
        out_type=jax.ShapeDtypeStruct((n_rows, d), jnp.int32),
        scratch_types=[
            pltpu.VMEM((chunks_per_tile, CHUNK), jnp.int32),
            pltpu.VMEM((2, GROUP_ROWS, d), jnp.int32),
            pltpu.SemaphoreType.DMA,
            pltpu.SemaphoreType.DMA,
        ],
        compiler_params=pltpu.CompilerParams(use_tc_tiling_on_sc=False),
    )
    def gather_kernel(idx_hbm, table_hbm, out_hbm, idx_v, rows_v, sem_g, sem_o):
        wid = lax.axis_index("s") * NUM_SC + lax.axis_index("c")
        chunk_base = wid * chunks_per_tile
        row_base = wid * rows_per_tile
        pltpu.sync_copy(idx_hbm.at[pl.ds(chunk_base, chunks_per_tile)], idx_v)

        def fire(g):
            buf = rows_v.at[g % 2]
            for i in range(GROUP):
                pltpu.async_copy(
                    table_hbm.at[idx_v.at[g * GROUP + i]],
                    buf.at[pl.ds(i * CHUNK, CHUNK)],
                    sem_g,
                )

        def out_slice(g):
            return out_hbm.at[pl.ds(row_base + g * GROUP_ROWS, GROUP_ROWS)]

        def drain_gathers(g):
            pltpu.make_async_copy(out_slice(g), rows_v.at[g % 2], sem_g).wait()

        def start_out(g):
            pltpu.async_copy(rows_v.at[g % 2], out_slice(g), sem_o)

        def drain_out(g):
            pltpu.make_async_copy(rows_v.at[g % 2], out_slice(g), sem_o).wait()

        fire(0)

        def body(g, carry):
            drain_gathers(g - 1)
            start_out(g - 1)

            @pl.when(g >= 2)
            def _():
                drain_out(g - 2)

            fire(g)
            return carry

        lax.fori_loop(1, groups_per_tile, body, 0)

        last = groups_per_tile - 1
        drain_gathers(last)
        start_out(last)
        drain_out(last - 1)
        drain_out(last)

    return gather_kernel


def _mlp_body(x_ref, w1_ref, b1_ref, w2_ref, b2_ref, o_ref):
    x = x_ref[...]
    h = jnp.dot(x, w1_ref[...], preferred_element_type=jnp.float32) + b1_ref[...]
    h = jnp.maximum(h, 0.0)
    o = jnp.dot(h, w2_ref[...], preferred_element_type=jnp.float32) + b2_ref[...]
    m = jnp.max(o, axis=1, keepdims=True)
    s = o - m
    lse = jnp.log(jnp.sum(jnp.exp(s), axis=1, keepdims=True))
    o_ref[...] = s - lse


def kernel(inputs, embed_table, W1, b1, W2, b2):
    batch, ctx = inputs.shape
    n_rows = batch * ctx
    feat = ctx * DIM_EMB
    d_words = DIM_EMB // 2

    table_bf = embed_table.astype(jnp.bfloat16)
    table_i32 = jax.lax.bitcast_convert_type(
        table_bf.reshape(-1, d_words, 2), jnp.int32
    )

    idx2d = inputs.reshape(-1, CHUNK).astype(jnp.int32)
    emb_i32 = _make_gather(n_rows, d_words)(idx2d, table_i32)
    embds = jax.lax.bitcast_convert_type(emb_i32, jnp.bfloat16).reshape(batch, feat)

    tb = 512
    out = pl.pallas_call(
        _mlp_body,
        grid=(batch // tb,),
        in_specs=[
            pl.BlockSpec((tb, feat), lambda i: (i, 0)),
            pl.BlockSpec((feat, DIM_HID), lambda i: (0, 0)),
            pl.BlockSpec((1, DIM_HID), lambda i: (0, 0)),
            pl.BlockSpec((DIM_HID, DIM_OUT), lambda i: (0, 0)),
            pl.BlockSpec((1, DIM_OUT), lambda i: (0, 0)),
        ],
        out_specs=pl.BlockSpec((tb, DIM_OUT), lambda i: (i, 0)),
        out_shape=jax.ShapeDtypeStruct((batch, DIM_OUT), jnp.float32),
    )(embds, W1.astype(jnp.bfloat16), b1.reshape(1, DIM_HID), W2, b2.reshape(1, DIM_OUT))
    return out


# speedup vs baseline: 28.9783x; 28.9783x over previous
import functools

import jax
import jax.numpy as jnp
from jax import lax
from jax.experimental import pallas as pl
from jax.experimental.pallas import tpu as pltpu
from jax.experimental.pallas import tpu_sc as plsc

DIM_EMB = 32
DIM_HID = 64
DIM_OUT = 2

NUM_SC = 2
NUM_SUBCORES = 16
NW = NUM_SC * NUM_SUBCORES
CHUNK = 128
GROUP = 8
GROUP_ROWS = GROUP * CHUNK


def _make_gather(n_rows: int, d: int):
    assert n_rows % (NW * GROUP_ROWS) == 0
    rows_per_tile = n_rows // NW
    chunks_per_tile = rows_per_tile // CHUNK
    groups_per_tile = chunks_per_tile // GROUP

    mesh = plsc.VectorSubcoreMesh(core_axis_name="c", subcore_axis_name="s")

    @functools.partial(
        pl.kernel,
        mesh=mesh,
        out_type=jax.ShapeDtypeStruct((n_rows, d), jnp.bfloat16),
        scratch_types=[
            pltpu.VMEM((chunks_per_tile, CHUNK), jnp.int32),
            pltpu.VMEM((2, GROUP_ROWS, d), jnp.bfloat16),
            pltpu.SemaphoreType.DMA,
            pltpu.SemaphoreType.DMA,
        ],
        compiler_params=pltpu.CompilerParams(use_tc_tiling_on_sc=False),
    )
    def gather_kernel(idx_hbm, table_hbm, out_hbm, idx_v, rows_v, sem_g, sem_o):
        wid = lax.axis_index("s") * NUM_SC + lax.axis_index("c")
        chunk_base = wid * chunks_per_tile
        row_base = wid * rows_per_tile
        pltpu.sync_copy(idx_hbm.at[pl.ds(chunk_base, chunks_per_tile)], idx_v)

        def fire(g):
            buf = rows_v.at[g % 2]
            for i in range(GROUP):
                pltpu.async_copy(
                    table_hbm.at[idx_v.at[g * GROUP + i]],
                    buf.at[pl.ds(i * CHUNK, CHUNK)],
                    sem_g,
                )

        def out_slice(g):
            return out_hbm.at[pl.ds(row_base + g * GROUP_ROWS, GROUP_ROWS)]

        def drain_gathers(g):
            pltpu.make_async_copy(out_slice(g), rows_v.at[g % 2], sem_g).wait()

        def start_out(g):
            pltpu.async_copy(rows_v.at[g % 2], out_slice(g), sem_o)

        def drain_out(g):
            pltpu.make_async_copy(rows_v.at[g % 2], out_slice(g), sem_o).wait()

        fire(0)

        def body(g, carry):
            drain_gathers(g - 1)
            start_out(g - 1)

            @pl.when(g >= 2)
            def _():
                drain_out(g - 2)

            fire(g)
            return carry

        lax.fori_loop(1, groups_per_tile, body, 0)

        last = groups_per_tile - 1
        drain_gathers(last)
        start_out(last)
        drain_out(last - 1)
        drain_out(last)

    return gather_kernel


def _mlp_body(x_ref, w1_ref, b1_ref, w2_ref, b2_ref, o_ref):
    x = x_ref[...]
    h = jnp.dot(x, w1_ref[...], preferred_element_type=jnp.float32) + b1_ref[...]
    h = jnp.maximum(h, 0.0)
    o = jnp.dot(h, w2_ref[...], preferred_element_type=jnp.float32) + b2_ref[...]
    m = jnp.max(o, axis=1, keepdims=True)
    s = o - m
    lse = jnp.log(jnp.sum(jnp.exp(s), axis=1, keepdims=True))
    o_ref[...] = s - lse


def kernel(inputs, embed_table, W1, b1, W2, b2):
    batch, ctx = inputs.shape
    n_rows = batch * ctx
    feat = ctx * DIM_EMB
    table_bf = embed_table.astype(jnp.bfloat16)
    idx2d = inputs.reshape(-1, CHUNK).astype(jnp.int32)
    embds = _make_gather(n_rows, DIM_EMB)(idx2d, table_bf).reshape(batch, feat)

    tb = 512
    out = pl.pallas_call(
        _mlp_body,
        grid=(batch // tb,),
        in_specs=[
            pl.BlockSpec((tb, feat), lambda i: (i, 0)),
            pl.BlockSpec((feat, DIM_HID), lambda i: (0, 0)),
            pl.BlockSpec((1, DIM_HID), lambda i: (0, 0)),
            pl.BlockSpec((DIM_HID, DIM_OUT), lambda i: (0, 0)),
            pl.BlockSpec((1, DIM_OUT), lambda i: (0, 0)),
        ],
        out_specs=pl.BlockSpec((tb, DIM_OUT), lambda i: (i, 0)),
        out_shape=jax.ShapeDtypeStruct((batch, DIM_OUT), jnp.float32),
    )(embds, W1.astype(jnp.bfloat16), b1.reshape(1, DIM_HID), W2, b2.reshape(1, DIM_OUT))
    return out
